# baseline (speedup 1.0000x reference)
import functools

import jax
import jax.numpy as jnp
from jax.experimental import pallas as pl
from jax.experimental.pallas import tpu as pltpu

N = 8192
D = 32
K = 20
ALPHA = 3.0
BLOCK = 256
NEG = -3.4e38
INF = 3.4e38


def _proj_kernel(e1_ref, e1t_ref, e2_ref, e2t_ref, w1_ref, b1_ref,
                 w2_ref, b2_ref, c1_ref, c2_ref):
    w1t = w1_ref[...].T
    w2t = w2_ref[...].T
    t1 = jnp.tanh(ALPHA * (jnp.dot(e1_ref[...], w1t,
                                   preferred_element_type=jnp.float32)
                           + b1_ref[...][None, :]))
    t2 = jnp.tanh(ALPHA * (jnp.dot(e2_ref[...], w2t,
                                   preferred_element_type=jnp.float32)
                           + b2_ref[...][None, :]))
    t1t = jnp.tanh(ALPHA * (jnp.dot(w1_ref[...], e1t_ref[...],
                                    preferred_element_type=jnp.float32)
                            + b1_ref[...][:, None]))
    t2t = jnp.tanh(ALPHA * (jnp.dot(w2_ref[...], e2t_ref[...],
                                    preferred_element_type=jnp.float32)
                            + b2_ref[...][:, None]))
    c1_ref[:, 0:D] = t1
    c1_ref[:, D:2 * D] = t2
    c2_ref[0:D, :] = t2t
    c2_ref[D:2 * D, :] = -t1t


def _adj_kernel(c1_ref, c2_ref, out_ref):
    a = jnp.dot(c1_ref[...], c2_ref[...],
                preferred_element_type=jnp.float32)

    ar = a.reshape(BLOCK, N // 128, 128)
    neg = jnp.full((BLOCK, 128), NEG, jnp.float32)
    s1, s2, s3, s4, s5 = ar[:, 0, :], neg, neg, neg, neg
    for v in range(1, N // 128):
        x = ar[:, v, :]
        r = jnp.minimum(s1, x)
        s1 = jnp.maximum(s1, x)
        r2 = jnp.minimum(s2, r)
        s2 = jnp.maximum(s2, r)
        r3 = jnp.minimum(s3, r2)
        s3 = jnp.maximum(s3, r2)
        r4 = jnp.minimum(s4, r3)
        s4 = jnp.maximum(s4, r3)
        s5 = jnp.maximum(s5, r4)
    summ = jnp.concatenate([s1, s2, s3, s4, s5], axis=1)

    def d20_body(_, carry):
        w, _t = carry
        m = jnp.max(w, axis=1, keepdims=True)
        w = jnp.where(w >= m, NEG, w)
        return w, m

    _, t = jax.lax.fori_loop(
        0, K, d20_body, (summ, jnp.zeros((BLOCK, 1), jnp.float32)))

    kf = float(K)
    c = jnp.sum(jnp.where(summ >= t, 1.0, 0.0), axis=1, keepdims=True)
    danger = jnp.max(jnp.where(s5 >= t, 1.0, 0.0), axis=1, keepdims=True)
    c = c + danger

    def raise_cond(carry):
        _t, c = carry
        return jnp.any(c > kf)

    def raise_body(carry):
        t, c = carry
        tn = jnp.min(jnp.where(a > t, a, INF), axis=1, keepdims=True)
        cn = jnp.sum(jnp.where(a >= tn, 1.0, 0.0), axis=1, keepdims=True)
        upd = jnp.logical_and(c > kf, cn >= kf)
        t = jnp.where(upd, tn, t)
        c = jnp.where(c > kf, jnp.where(cn >= kf, cn, kf), c)
        return t, c

    t, c = jax.lax.while_loop(raise_cond, raise_body, (t, c))
    out_ref[...] = jnp.where(a >= t, jnp.maximum(jnp.tanh(ALPHA * a), 0.0), 0.0)


@jax.jit
def kernel(idx, emb1_w, emb2_w, W1, b1, W2, b2):
    e1 = jnp.take(emb1_w, idx, axis=0)
    e2 = jnp.take(emb2_w, idx, axis=0)
    e1t = e1.T
    e2t = e2.T

    c1, c2 = pl.pallas_call(
        _proj_kernel,
        out_shape=(
            jax.ShapeDtypeStruct((N, 2 * D), jnp.float32),
            jax.ShapeDtypeStruct((2 * D, N), jnp.float32),
        ),
    )(e1, e1t, e2, e2t, W1, b1, W2, b2)

    grid = N // BLOCK
    out = pl.pallas_call(
        _adj_kernel,
        grid=(grid,),
        in_specs=[
            pl.BlockSpec((BLOCK, 2 * D), lambda i: (i, 0)),
            pl.BlockSpec((2 * D, N), lambda i: (0, 0)),
        ],
        out_specs=pl.BlockSpec((BLOCK, N), lambda i: (i, 0)),
        out_shape=jax.ShapeDtypeStruct((N, N), jnp.float32),
        compiler_params=pltpu.CompilerParams(
            dimension_semantics=("parallel",),
        ),
    )(c1, c2)
    return out


# speedup vs baseline: 2.7296x; 2.7296x over previous
import functools

import jax
import jax.numpy as jnp
from jax.experimental import pallas as pl
from jax.experimental.pallas import tpu as pltpu

N = 8192
D = 32
K = 20
ALPHA = 3.0
BLOCK = 256
NEG = -3.4e38
INF = 3.4e38


def _proj_kernel(e1_ref, e1t_ref, e2_ref, e2t_ref, w1_ref, b1_ref,
                 w2_ref, b2_ref, c1_ref, c2_ref):
    w1t = w1_ref[...].T
    w2t = w2_ref[...].T
    t1 = jnp.tanh(ALPHA * (jnp.dot(e1_ref[...], w1t,
                                   preferred_element_type=jnp.float32)
                           + b1_ref[...][None, :]))
    t2 = jnp.tanh(ALPHA * (jnp.dot(e2_ref[...], w2t,
                                   preferred_element_type=jnp.float32)
                           + b2_ref[...][None, :]))
    t1t = jnp.tanh(ALPHA * (jnp.dot(w1_ref[...], e1t_ref[...],
                                    preferred_element_type=jnp.float32)
                            + b1_ref[...][:, None]))
    t2t = jnp.tanh(ALPHA * (jnp.dot(w2_ref[...], e2t_ref[...],
                                    preferred_element_type=jnp.float32)
                            + b2_ref[...][:, None]))
    c1_ref[:, 0:D] = t1
    c1_ref[:, D:2 * D] = t2
    c2_ref[0:D, :] = t2t
    c2_ref[D:2 * D, :] = -t1t


def _adj_kernel(c1_ref, c2_ref, out_ref):
    a = jnp.dot(c1_ref[...], c2_ref[...],
                preferred_element_type=jnp.float32)

    ar = a.reshape(BLOCK, N // 128, 128)
    m1 = jnp.max(ar, axis=1)
    m2 = jnp.max(jnp.where(ar < m1[:, None, :], ar, NEG), axis=1)
    m3 = jnp.max(jnp.where(ar < m2[:, None, :], ar, NEG), axis=1)
    summ = jnp.concatenate([m1, m2, m3], axis=1)

    summ_t = summ.T

    def d20_body(_, carry):
        w, _t = carry
        m = jnp.max(w, axis=0, keepdims=True)
        w = jnp.where(w >= m, NEG, w)
        return w, m

    _, t_row = jax.lax.fori_loop(
        0, K, d20_body, (summ_t, jnp.zeros((1, BLOCK), jnp.float32)))
    t = t_row.T

    kf = float(K)
    c = jnp.sum(jnp.where(summ >= t, 1.0, 0.0), axis=1, keepdims=True)
    danger = jnp.max(jnp.where(m3 >= t, 1.0, 0.0), axis=1, keepdims=True)
    c = c + danger

    def raise_cond(carry):
        _t, c = carry
        return jnp.any(c > kf)

    def raise_body(carry):
        t, c = carry
        tn = jnp.min(jnp.where(a > t, a, INF), axis=1, keepdims=True)
        cn = jnp.sum(jnp.where(a >= tn, 1.0, 0.0), axis=1, keepdims=True)
        upd = jnp.logical_and(c > kf, cn >= kf)
        t = jnp.where(upd, tn, t)
        c = jnp.where(c > kf, jnp.where(cn >= kf, cn, kf), c)
        return t, c

    t, c = jax.lax.while_loop(raise_cond, raise_body, (t, c))
    out_ref[...] = jnp.where(a >= t, jnp.maximum(jnp.tanh(ALPHA * a), 0.0), 0.0)


@jax.jit
def kernel(idx, emb1_w, emb2_w, W1, b1, W2, b2):
    e1 = jnp.take(emb1_w, idx, axis=0)
    e2 = jnp.take(emb2_w, idx, axis=0)
    e1t = e1.T
    e2t = e2.T

    c1, c2 = pl.pallas_call(
        _proj_kernel,
        out_shape=(
            jax.ShapeDtypeStruct((N, 2 * D), jnp.float32),
            jax.ShapeDtypeStruct((2 * D, N), jnp.float32),
        ),
    )(e1, e1t, e2, e2t, W1, b1, W2, b2)

    grid = N // BLOCK
    out = pl.pallas_call(
        _adj_kernel,
        grid=(grid,),
        in_specs=[
            pl.BlockSpec((BLOCK, 2 * D), lambda i: (i, 0)),
            pl.BlockSpec((2 * D, N), lambda i: (0, 0)),
        ],
        out_specs=pl.BlockSpec((BLOCK, N), lambda i: (i, 0)),
        out_shape=jax.ShapeDtypeStruct((N, N), jnp.float32),
        compiler_params=pltpu.CompilerParams(
            dimension_semantics=("parallel",),
        ),
    )(c1, c2)
    return out


# speedup vs baseline: 3.2512x; 1.1911x over previous
import functools

import jax
import jax.numpy as jnp
from jax.experimental import pallas as pl
from jax.experimental.pallas import tpu as pltpu

N = 8192
D = 32
K = 20
ALPHA = 3.0
BLOCK = 256
NEG = -3.4e38
INF = 3.4e38


def _proj_kernel(e1_ref, e1t_ref, e2_ref, e2t_ref, w1_ref, b1_ref,
                 w2_ref, b2_ref, c1_ref, c2_ref):
    w1t = w1_ref[...].T
    w2t = w2_ref[...].T
    t1 = jnp.tanh(ALPHA * (jnp.dot(e1_ref[...], w1t,
                                   preferred_element_type=jnp.float32)
                           + b1_ref[...][None, :]))
    t2 = jnp.tanh(ALPHA * (jnp.dot(e2_ref[...], w2t,
                                   preferred_element_type=jnp.float32)
                           + b2_ref[...][None, :]))
    t1t = jnp.tanh(ALPHA * (jnp.dot(w1_ref[...], e1t_ref[...],
                                    preferred_element_type=jnp.float32)
                            + b1_ref[...][:, None]))
    t2t = jnp.tanh(ALPHA * (jnp.dot(w2_ref[...], e2t_ref[...],
                                    preferred_element_type=jnp.float32)
                            + b2_ref[...][:, None]))
    c1_ref[:, 0:D] = t1
    c1_ref[:, D:2 * D] = t2
    c2_ref[0:D, :] = t2t
    c2_ref[D:2 * D, :] = -t1t


def _adj_kernel(c1_ref, c2_ref, out_ref):
    a = jnp.dot(c1_ref[...], c2_ref[...],
                preferred_element_type=jnp.float32)

    ar = a.reshape(BLOCK, N // 128, 128)
    m1 = jnp.max(ar, axis=1)
    m2 = jnp.max(jnp.where(ar < m1[:, None, :], ar, NEG), axis=1)
    summ = jnp.concatenate([m1, m2], axis=1)

    summ_t = summ.T
    reps = summ_t.shape[0] // 8

    def _tree_max_rep(w):
        x = w
        while x.shape[0] > 8:
            h = x.shape[0] // 2
            x = jnp.maximum(x[:h], x[h:])
        for sh in (4, 2, 1):
            x = jnp.maximum(x, jnp.roll(x, sh, axis=0))
        return x

    def d20_body(_, t_rep):
        masked = jnp.where(summ_t < jnp.tile(t_rep, (reps, 1)), summ_t, NEG)
        return _tree_max_rep(masked)

    t_rep = jax.lax.fori_loop(
        0, K, d20_body, jnp.full((8, BLOCK), INF, jnp.float32))
    t = t_rep[0:1, :].T

    kf = float(K)
    c = jnp.sum(jnp.where(a >= t, 1.0, 0.0), axis=1, keepdims=True)

    def raise_cond(carry):
        _t, c = carry
        return jnp.any(c > kf)

    def raise_body(carry):
        t, c = carry
        tn = jnp.min(jnp.where(a > t, a, INF), axis=1, keepdims=True)
        cn = jnp.sum(jnp.where(a >= tn, 1.0, 0.0), axis=1, keepdims=True)
        upd = jnp.logical_and(c > kf, cn >= kf)
        t = jnp.where(upd, tn, t)
        c = jnp.where(c > kf, jnp.where(cn >= kf, cn, kf), c)
        return t, c

    t, c = jax.lax.while_loop(raise_cond, raise_body, (t, c))
    out_ref[...] = jnp.where(a >= t, jnp.maximum(jnp.tanh(ALPHA * a), 0.0), 0.0)


@jax.jit
def kernel(idx, emb1_w, emb2_w, W1, b1, W2, b2):
    e1 = jnp.take(emb1_w, idx, axis=0)
    e2 = jnp.take(emb2_w, idx, axis=0)
    e1t = e1.T
    e2t = e2.T

    c1, c2 = pl.pallas_call(
        _proj_kernel,
        out_shape=(
            jax.ShapeDtypeStruct((N, 2 * D), jnp.float32),
            jax.ShapeDtypeStruct((2 * D, N), jnp.float32),
        ),
    )(e1, e1t, e2, e2t, W1, b1, W2, b2)

    grid = N // BLOCK
    out = pl.pallas_call(
        _adj_kernel,
        grid=(grid,),
        in_specs=[
            pl.BlockSpec((BLOCK, 2 * D), lambda i: (i, 0)),
            pl.BlockSpec((2 * D, N), lambda i: (0, 0)),
        ],
        out_specs=pl.BlockSpec((BLOCK, N), lambda i: (i, 0)),
        out_shape=jax.ShapeDtypeStruct((N, N), jnp.float32),
        compiler_params=pltpu.CompilerParams(
            dimension_semantics=("parallel",),
        ),
    )(c1, c2)
    return out
